# baseline (speedup 1.0000x reference)
import jax
import jax.numpy as jnp
from jax.experimental import pallas as pl
from jax.experimental.pallas import tpu as pltpu

B = 4096
S = 8
D = 512
P = 256
EMB = 64
NE = 7

BB = 512


def _gru_block(states_ref, ids_ref, du_ref, emo_ref, emb_ref, w_ih_ref,
               w_hh_ref, b_ih_ref, b_hh_ref, out_ref):
    ids3 = ids_ref[...]
    emo = emo_ref[...]

    st = states_ref[...]
    iota_s = jax.lax.broadcasted_iota(jnp.int32, (BB, S, D), 1)
    mask3 = ids3 == iota_s
    h_old = jnp.sum(jnp.where(mask3, st, 0.0), axis=1)

    emask = (emo == jax.lax.broadcasted_iota(jnp.int32, (BB, NE + 1), 1))
    other_emb = jax.lax.dot_general(
        emask.astype(jnp.float32), emb_ref[...],
        (((1,), (0,)), ((), ())), preferred_element_type=jnp.float32)

    gi = jax.lax.dot_general(du_ref[...], w_ih_ref[:, :P],
                             (((1,), (1,)), ((), ())),
                             preferred_element_type=jnp.float32)
    gi += jax.lax.dot_general(other_emb, w_ih_ref[:, P:],
                              (((1,), (1,)), ((), ())),
                              preferred_element_type=jnp.float32)
    gi += b_ih_ref[...]
    gh = jax.lax.dot_general(h_old, w_hh_ref[...],
                             (((1,), (1,)), ((), ())),
                             preferred_element_type=jnp.float32)
    gh += b_hh_ref[...]

    r = jax.nn.sigmoid(gi[:, :D] + gh[:, :D])
    z = jax.nn.sigmoid(gi[:, D:2 * D] + gh[:, D:2 * D])
    n = jnp.tanh(gi[:, 2 * D:] + r * gh[:, 2 * D:])
    h_new = (1.0 - z) * n + z * h_old

    h_new3 = jax.lax.broadcast_in_dim(h_new, (BB, S, D), (0, 2))
    out_ref[...] = jnp.where(mask3, h_new3, st)


def kernel(states, speaker_ids, delta_u, other_emo_ids, emb_table, w_ih,
           w_hh, b_ih, b_hh):
    ids3 = jnp.clip(speaker_ids, 0, S - 1).astype(jnp.int32).reshape(B, 1, 1)
    emo2 = other_emo_ids.astype(jnp.int32).reshape(B, 1)
    b_ih2 = b_ih.reshape(1, 3 * D)
    b_hh2 = b_hh.reshape(1, 3 * D)

    grid = (B // BB,)
    out = pl.pallas_call(
        _gru_block,
        grid=grid,
        in_specs=[
            pl.BlockSpec((BB, S, D), lambda i: (i, 0, 0)),
            pl.BlockSpec((BB, 1, 1), lambda i: (i, 0, 0)),
            pl.BlockSpec((BB, P), lambda i: (i, 0)),
            pl.BlockSpec((BB, 1), lambda i: (i, 0)),
            pl.BlockSpec((NE + 1, EMB), lambda i: (0, 0)),
            pl.BlockSpec((3 * D, P + EMB), lambda i: (0, 0)),
            pl.BlockSpec((3 * D, D), lambda i: (0, 0)),
            pl.BlockSpec((1, 3 * D), lambda i: (0, 0)),
            pl.BlockSpec((1, 3 * D), lambda i: (0, 0)),
        ],
        out_specs=pl.BlockSpec((BB, S, D), lambda i: (i, 0, 0)),
        out_shape=jax.ShapeDtypeStruct((B, S, D), states.dtype),
        compiler_params=pltpu.CompilerParams(
            dimension_semantics=("parallel",)),
    )(states, ids3, delta_u, emo2, emb_table, w_ih, w_hh, b_ih2, b_hh2)
    return out


# speedup vs baseline: 1.0164x; 1.0164x over previous
import jax
import jax.numpy as jnp
from jax.experimental import pallas as pl
from jax.experimental.pallas import tpu as pltpu

B = 4096
S = 8
D = 512
P = 256
EMB = 64
NE = 7

BB = 512


def _gru_block(states_ref, ids_ref, du_ref, emo_ref, emb_ref, w_ih_ref,
               w_hh_ref, b_ih_ref, b_hh_ref, out_ref):
    ids3 = ids_ref[...]
    emo = emo_ref[...]

    st = states_ref[...]
    iota_s = jax.lax.broadcasted_iota(jnp.int32, (BB, S, D), 1)
    mask3 = ids3 == iota_s
    h_old = jnp.sum(jnp.where(mask3, st, 0.0), axis=1)

    emask = (emo == jax.lax.broadcasted_iota(jnp.int32, (BB, NE + 1), 1))
    other_emb = jax.lax.dot_general(
        emask.astype(jnp.float32), emb_ref[...],
        (((1,), (0,)), ((), ())), preferred_element_type=jnp.float32)

    inp = jnp.concatenate([du_ref[...], other_emb], axis=1)
    gi = jax.lax.dot_general(inp, w_ih_ref[...],
                             (((1,), (1,)), ((), ())),
                             preferred_element_type=jnp.float32)
    gi += b_ih_ref[...]
    gh = jax.lax.dot_general(h_old, w_hh_ref[...],
                             (((1,), (1,)), ((), ())),
                             preferred_element_type=jnp.float32)
    gh += b_hh_ref[...]

    r = jax.nn.sigmoid(gi[:, :D] + gh[:, :D])
    z = jax.nn.sigmoid(gi[:, D:2 * D] + gh[:, D:2 * D])
    n = jnp.tanh(gi[:, 2 * D:] + r * gh[:, 2 * D:])
    h_new = (1.0 - z) * n + z * h_old

    h_new3 = jax.lax.broadcast_in_dim(h_new, (BB, S, D), (0, 2))
    out_ref[...] = jnp.where(mask3, h_new3, st)


def kernel(states, speaker_ids, delta_u, other_emo_ids, emb_table, w_ih,
           w_hh, b_ih, b_hh):
    ids3 = jnp.clip(speaker_ids, 0, S - 1).astype(jnp.int32).reshape(B, 1, 1)
    emo2 = other_emo_ids.astype(jnp.int32).reshape(B, 1)
    b_ih2 = b_ih.reshape(1, 3 * D)
    b_hh2 = b_hh.reshape(1, 3 * D)

    grid = (B // BB,)
    out = pl.pallas_call(
        _gru_block,
        grid=grid,
        in_specs=[
            pl.BlockSpec((BB, S, D), lambda i: (i, 0, 0)),
            pl.BlockSpec((BB, 1, 1), lambda i: (i, 0, 0)),
            pl.BlockSpec((BB, P), lambda i: (i, 0)),
            pl.BlockSpec((BB, 1), lambda i: (i, 0)),
            pl.BlockSpec((NE + 1, EMB), lambda i: (0, 0)),
            pl.BlockSpec((3 * D, P + EMB), lambda i: (0, 0)),
            pl.BlockSpec((3 * D, D), lambda i: (0, 0)),
            pl.BlockSpec((1, 3 * D), lambda i: (0, 0)),
            pl.BlockSpec((1, 3 * D), lambda i: (0, 0)),
        ],
        out_specs=pl.BlockSpec((BB, S, D), lambda i: (i, 0, 0)),
        out_shape=jax.ShapeDtypeStruct((B, S, D), states.dtype),
        compiler_params=pltpu.CompilerParams(
            dimension_semantics=("parallel",)),
    )(states, ids3, delta_u, emo2, emb_table, w_ih, w_hh, b_ih2, b_hh2)
    return out


# speedup vs baseline: 1.1228x; 1.1047x over previous
import jax
import jax.numpy as jnp
from jax.experimental import pallas as pl
from jax.experimental.pallas import tpu as pltpu

B = 4096
S = 8
D = 512
P = 256
EMB = 64
NE = 7

BB = 512


def _gru_block(states_ref, ids_ref, du_ref, emo_ref, emb_ref, w_ih_ref,
               w_hh_ref, b_ih_ref, b_hh_ref, out_ref, h_scr_ref):
    ids3 = ids_ref[...]
    emo = emo_ref[...]

    st = states_ref[...]
    iota_s = jax.lax.broadcasted_iota(jnp.int32, (BB, S, D), 1)
    mask3 = ids3 == iota_s
    CH = 128
    ids2 = ids3[:, :, 0]
    for c in range(BB // CH):
        tgt = (S * jax.lax.broadcasted_iota(jnp.int32, (CH, 1), 0)
               + ids2[c * CH:(c + 1) * CH])
        selc = (tgt == jax.lax.broadcasted_iota(jnp.int32, (CH, CH * S), 1))
        h_scr_ref[c * CH:(c + 1) * CH, :] = jax.lax.dot_general(
            selc.astype(jnp.float32),
            st[c * CH:(c + 1) * CH].reshape(CH * S, D),
            (((1,), (0,)), ((), ())), preferred_element_type=jnp.float32)
    h_old = h_scr_ref[...]

    emask = (emo == jax.lax.broadcasted_iota(jnp.int32, (BB, NE + 1), 1))
    other_emb = jax.lax.dot_general(
        emask.astype(jnp.float32), emb_ref[...],
        (((1,), (0,)), ((), ())), preferred_element_type=jnp.float32)

    inp = jnp.concatenate([du_ref[...], other_emb], axis=1)
    gi = jax.lax.dot_general(inp, w_ih_ref[...],
                             (((1,), (1,)), ((), ())),
                             preferred_element_type=jnp.float32)
    gi += b_ih_ref[...]
    gh = jax.lax.dot_general(h_old, w_hh_ref[...],
                             (((1,), (1,)), ((), ())),
                             preferred_element_type=jnp.float32)
    gh += b_hh_ref[...]

    r = jax.nn.sigmoid(gi[:, :D] + gh[:, :D])
    z = jax.nn.sigmoid(gi[:, D:2 * D] + gh[:, D:2 * D])
    n = jnp.tanh(gi[:, 2 * D:] + r * gh[:, 2 * D:])
    h_new = (1.0 - z) * n + z * h_old

    h_new3 = jax.lax.broadcast_in_dim(h_new, (BB, S, D), (0, 2))
    out_ref[...] = jnp.where(mask3, h_new3, st)


def kernel(states, speaker_ids, delta_u, other_emo_ids, emb_table, w_ih,
           w_hh, b_ih, b_hh):
    ids3 = jnp.clip(speaker_ids, 0, S - 1).astype(jnp.int32).reshape(B, 1, 1)
    emo2 = other_emo_ids.astype(jnp.int32).reshape(B, 1)
    b_ih2 = b_ih.reshape(1, 3 * D)
    b_hh2 = b_hh.reshape(1, 3 * D)

    grid = (B // BB,)
    out = pl.pallas_call(
        _gru_block,
        grid=grid,
        in_specs=[
            pl.BlockSpec((BB, S, D), lambda i: (i, 0, 0)),
            pl.BlockSpec((BB, 1, 1), lambda i: (i, 0, 0)),
            pl.BlockSpec((BB, P), lambda i: (i, 0)),
            pl.BlockSpec((BB, 1), lambda i: (i, 0)),
            pl.BlockSpec((NE + 1, EMB), lambda i: (0, 0)),
            pl.BlockSpec((3 * D, P + EMB), lambda i: (0, 0)),
            pl.BlockSpec((3 * D, D), lambda i: (0, 0)),
            pl.BlockSpec((1, 3 * D), lambda i: (0, 0)),
            pl.BlockSpec((1, 3 * D), lambda i: (0, 0)),
        ],
        out_specs=pl.BlockSpec((BB, S, D), lambda i: (i, 0, 0)),
        out_shape=jax.ShapeDtypeStruct((B, S, D), states.dtype),
        compiler_params=pltpu.CompilerParams(
            dimension_semantics=("parallel",)),
        scratch_shapes=[pltpu.VMEM((BB, D), jnp.float32)],
    )(states, ids3, delta_u, emo2, emb_table, w_ih, w_hh, b_ih2, b_hh2)
    return out
